# baseline (speedup 1.0000x reference)
import jax
import jax.numpy as jnp
from jax.experimental import pallas as pl
from jax.experimental.pallas import tpu as pltpu

N = 100000
F = 128
FG = 256
H = 8
C = 16
BLK = 4000
NB = N // BLK


def _ln(x, scale, bias, eps=1e-5):
    mu = jnp.mean(x, axis=-1, keepdims=True)
    var = jnp.mean((x - mu) * (x - mu), axis=-1, keepdims=True)
    return (x - mu) * jax.lax.rsqrt(var + eps) * scale + bias


def _dot(a, b):
    return jnp.dot(a, b, preferred_element_type=jnp.float32)


def _kernel(view_ref, sp_ref, g_ref,
            ln_g2v_s, ln_g2v_b, W_g2v, b_g2v,
            Wl_v, bl_v, Wr_v, br_v, AE_v, bb_v,
            ln_g2s_s, ln_g2s_b, W_g2s, b_g2s,
            Wl_s, bl_s, Wr_s, br_s, AE_s, bb_s,
            ln_pre_s, ln_pre_b, W_mlp, b_mlp,
            out_ref,
            m_v, s_v, w_v, xr_v, m_s, s_s, w_s, xr_s):
    i = pl.program_id(0)

    @pl.when(i == 0)
    def _init():
        g = g_ref[...]
        gv = jnp.maximum(_ln(g, ln_g2v_s[...], ln_g2v_b[...]), 0.0)
        xv = _dot(gv, W_g2v[...]) + b_g2v[...]
        xr_v[...] = bl_v[...] + _dot(xv, Wr_v[...]) + br_v[...]
        gs = jnp.maximum(_ln(g, ln_g2s_s[...], ln_g2s_b[...]), 0.0)
        xs = _dot(gs, W_g2s[...]) + b_g2s[...]
        xr_s[...] = bl_s[...] + _dot(xs, Wr_s[...]) + br_s[...]
        neg = jnp.full((1, F), -jnp.inf, jnp.float32)
        m_v[...] = neg
        m_s[...] = neg
        zero = jnp.zeros((1, F), jnp.float32)
        s_v[...] = zero
        s_s[...] = zero
        w_v[...] = zero
        w_s[...] = zero

    def stream(x_ref, Wl, xr_ref, AE, m_ref, s_ref, w_ref):
        x = x_ref[...]
        y = _dot(x, Wl[...])
        z = y + xr_ref[...]
        e = jnp.maximum(z, 0.2 * z)
        lb = _dot(e.astype(jnp.bfloat16), AE[...])
        m_old = m_ref[...]
        m_new = jnp.maximum(m_old, jnp.max(lb, axis=0, keepdims=True))
        corr = jnp.exp(m_old - m_new)
        pb = jnp.exp(lb - m_new)
        s_ref[...] = s_ref[...] * corr + jnp.sum(pb, axis=0, keepdims=True)
        w_ref[...] = w_ref[...] * corr + jnp.sum(pb * y, axis=0, keepdims=True)
        m_ref[...] = m_new

    stream(view_ref, Wl_v, xr_v, AE_v, m_v, s_v, w_v)
    stream(sp_ref, Wl_s, xr_s, AE_s, m_s, s_s, w_s)

    @pl.when(i == NB - 1)
    def _fin():
        v2g = w_v[...] / s_v[...] + bb_v[...]
        s2g = w_s[...] / s_s[...] + bb_s[...]
        x = g_ref[...] + jnp.concatenate([v2g, s2g], axis=1)
        y = jnp.maximum(_ln(x, ln_pre_s[...], ln_pre_b[...]), 0.0)
        y = _dot(y, W_mlp[...]) + b_mlp[...]
        out_ref[...] = x + y


def kernel(view_features, scenepoint_features, prev_global_features,
           ln_g2v_s, ln_g2v_b, W_g2v, b_g2v,
           Wl_v, bl_v, Wr_v, br_v, att_v, bias_v,
           ln_g2s_s, ln_g2s_b, W_g2s, b_g2s,
           Wl_s, bl_s, Wr_s, br_s, att_s, bias_s,
           ln_pre_s, ln_pre_b, W_mlp, b_mlp):
    row = lambda a: a.reshape(1, -1)
    heads = jnp.arange(F) // C
    same = (heads[:, None] == heads[None, :]).astype(jnp.float32)
    AE_v = same * att_v.reshape(-1)[:, None]
    AE_s = same * att_s.reshape(-1)[:, None]
    bb_v = row(bl_v + bias_v)
    bb_s = row(bl_s + bias_s)

    blk = pl.BlockSpec((BLK, F), lambda i: (i, 0))

    def full(shape):
        return pl.BlockSpec(shape, lambda i: (0,) * len(shape))

    ins = [
        view_features, scenepoint_features, prev_global_features,
        row(ln_g2v_s), row(ln_g2v_b), W_g2v, row(b_g2v),
        Wl_v, row(bl_v), Wr_v, row(br_v), AE_v.astype(jnp.bfloat16), bb_v,
        row(ln_g2s_s), row(ln_g2s_b), W_g2s, row(b_g2s),
        Wl_s, row(bl_s), Wr_s, row(br_s), AE_s.astype(jnp.bfloat16), bb_s,
        row(ln_pre_s), row(ln_pre_b), W_mlp, row(b_mlp),
    ]
    in_specs = [blk, blk] + [full(a.shape) for a in ins[2:]]

    scratch = [pltpu.VMEM((1, F), jnp.float32) for _ in range(8)]

    return pl.pallas_call(
        _kernel,
        grid=(NB,),
        in_specs=in_specs,
        out_specs=full((1, FG)),
        out_shape=jax.ShapeDtypeStruct((1, FG), jnp.float32),
        scratch_shapes=scratch,
        compiler_params=pltpu.CompilerParams(
            dimension_semantics=("arbitrary",)),
    )(*ins)


# speedup vs baseline: 1.0414x; 1.0414x over previous
import jax
import jax.numpy as jnp
from jax.experimental import pallas as pl
from jax.experimental.pallas import tpu as pltpu

N = 100000
F = 128
FG = 256
H = 8
C = 16
BLK = 4000
NB = N // BLK


def _ln(x, scale, bias, eps=1e-5):
    mu = jnp.mean(x, axis=-1, keepdims=True)
    var = jnp.mean((x - mu) * (x - mu), axis=-1, keepdims=True)
    return (x - mu) * jax.lax.rsqrt(var + eps) * scale + bias


def _dot(a, b):
    return jnp.dot(a, b, preferred_element_type=jnp.float32)


def _kernel(view_ref, sp_ref, g_ref,
            ln_g2v_s, ln_g2v_b, W_g2v, b_g2v,
            Wl_v, bl_v, Wr_v, br_v, AE_v, bb_v,
            ln_g2s_s, ln_g2s_b, W_g2s, b_g2s,
            Wl_s, bl_s, Wr_s, br_s, AE_s, bb_s,
            ln_pre_s, ln_pre_b, W_mlp, b_mlp,
            out_ref,
            m_v, s_v, w_v, xr_v, m_s, s_s, w_s, xr_s):
    i = pl.program_id(0)

    @pl.when(i == 0)
    def _init():
        g = g_ref[...]
        gv = jnp.maximum(_ln(g, ln_g2v_s[...], ln_g2v_b[...]), 0.0)
        xv = _dot(gv, W_g2v[...]) + b_g2v[...]
        xr_v[...] = bl_v[...] + _dot(xv, Wr_v[...]) + br_v[...]
        gs = jnp.maximum(_ln(g, ln_g2s_s[...], ln_g2s_b[...]), 0.0)
        xs = _dot(gs, W_g2s[...]) + b_g2s[...]
        xr_s[...] = bl_s[...] + _dot(xs, Wr_s[...]) + br_s[...]
        neg = jnp.full((1, F), -jnp.inf, jnp.float32)
        m_v[...] = neg
        m_s[...] = neg
        zero = jnp.zeros((1, F), jnp.float32)
        s_v[...] = zero
        s_s[...] = zero
        w_v[...] = zero
        w_s[...] = zero

    def stream(x_ref, Wl, xr_ref, AE, m_ref, s_ref, w_ref):
        x = x_ref[...]
        y = _dot(x, Wl[...])
        z = y + xr_ref[...]
        e = jnp.maximum(z, 0.2 * z)
        lb = _dot(e, AE[...])
        m_old = m_ref[...]
        m_new = jnp.maximum(m_old, jnp.max(lb, axis=0, keepdims=True))
        corr = jnp.exp2(m_old - m_new)
        pb = jnp.exp2(lb - m_new)
        s_ref[...] = s_ref[...] * corr + jnp.sum(pb, axis=0, keepdims=True)
        w_ref[...] = w_ref[...] * corr + jnp.sum(pb * y, axis=0, keepdims=True)
        m_ref[...] = m_new

    stream(view_ref, Wl_v, xr_v, AE_v, m_v, s_v, w_v)
    stream(sp_ref, Wl_s, xr_s, AE_s, m_s, s_s, w_s)

    @pl.when(i == NB - 1)
    def _fin():
        v2g = w_v[...] / s_v[...] + bb_v[...]
        s2g = w_s[...] / s_s[...] + bb_s[...]
        x = g_ref[...] + jnp.concatenate([v2g, s2g], axis=1)
        y = jnp.maximum(_ln(x, ln_pre_s[...], ln_pre_b[...]), 0.0)
        y = _dot(y, W_mlp[...]) + b_mlp[...]
        out_ref[...] = x + y


def kernel(view_features, scenepoint_features, prev_global_features,
           ln_g2v_s, ln_g2v_b, W_g2v, b_g2v,
           Wl_v, bl_v, Wr_v, br_v, att_v, bias_v,
           ln_g2s_s, ln_g2s_b, W_g2s, b_g2s,
           Wl_s, bl_s, Wr_s, br_s, att_s, bias_s,
           ln_pre_s, ln_pre_b, W_mlp, b_mlp):
    row = lambda a: a.reshape(1, -1)
    heads = jnp.arange(F) // C
    same = (heads[:, None] == heads[None, :]).astype(jnp.float32)
    log2e = 1.4426950408889634
    AE_v = same * (att_v.reshape(-1)[:, None] * log2e)
    AE_s = same * (att_s.reshape(-1)[:, None] * log2e)
    bb_v = row(bl_v + bias_v)
    bb_s = row(bl_s + bias_s)

    blk = pl.BlockSpec((BLK, F), lambda i: (i, 0))

    def full(shape):
        return pl.BlockSpec(shape, lambda i: (0,) * len(shape))

    ins = [
        view_features, scenepoint_features, prev_global_features,
        row(ln_g2v_s), row(ln_g2v_b), W_g2v, row(b_g2v),
        Wl_v, row(bl_v), Wr_v, row(br_v), AE_v, bb_v,
        row(ln_g2s_s), row(ln_g2s_b), W_g2s, row(b_g2s),
        Wl_s, row(bl_s), Wr_s, row(br_s), AE_s, bb_s,
        row(ln_pre_s), row(ln_pre_b), W_mlp, row(b_mlp),
    ]
    in_specs = [blk, blk] + [full(a.shape) for a in ins[2:]]

    scratch = [pltpu.VMEM((1, F), jnp.float32) for _ in range(8)]

    return pl.pallas_call(
        _kernel,
        grid=(NB,),
        in_specs=in_specs,
        out_specs=full((1, FG)),
        out_shape=jax.ShapeDtypeStruct((1, FG), jnp.float32),
        scratch_shapes=scratch,
        compiler_params=pltpu.CompilerParams(
            dimension_semantics=("arbitrary",)),
    )(*ins)


# speedup vs baseline: 1.1244x; 1.0797x over previous
import jax
import jax.numpy as jnp
from jax.experimental import pallas as pl
from jax.experimental.pallas import tpu as pltpu

N = 100000
F = 128
FG = 256
H = 8
C = 16
BLK = 4000
NB = N // BLK


def _ln(x, scale, bias, eps=1e-5):
    mu = jnp.mean(x, axis=-1, keepdims=True)
    var = jnp.mean((x - mu) * (x - mu), axis=-1, keepdims=True)
    return (x - mu) * jax.lax.rsqrt(var + eps) * scale + bias


def _dot(a, b):
    return jnp.dot(a, b, preferred_element_type=jnp.float32)


def _kernel(view_ref, sp_ref, g_ref,
            ln_g2v_s, ln_g2v_b, W_g2v, b_g2v,
            Wl_v, bl_v, Wr_v, br_v, AE_v, bb_v,
            ln_g2s_s, ln_g2s_b, W_g2s, b_g2s,
            Wl_s, bl_s, Wr_s, br_s, AE_s, bb_s,
            ln_pre_s, ln_pre_b, W_mlp, b_mlp,
            out_ref,
            m_v0, s_v0, w_v0, m_v1, s_v1, w_v1, xr_v,
            m_s0, s_s0, w_s0, m_s1, s_s1, w_s1, xr_s):
    i = pl.program_id(0)

    @pl.when(i == 0)
    def _init():
        g = g_ref[...]
        gv = jnp.maximum(_ln(g, ln_g2v_s[...], ln_g2v_b[...]), 0.0)
        xv = _dot(gv, W_g2v[...]) + b_g2v[...]
        xr_v[...] = bl_v[...] + _dot(xv, Wr_v[...]) + br_v[...]
        gs = jnp.maximum(_ln(g, ln_g2s_s[...], ln_g2s_b[...]), 0.0)
        xs = _dot(gs, W_g2s[...]) + b_g2s[...]
        xr_s[...] = bl_s[...] + _dot(xs, Wr_s[...]) + br_s[...]
        neg = jnp.full((1, F), -jnp.inf, jnp.float32)
        zero = jnp.zeros((1, F), jnp.float32)
        for r in (m_v0, m_v1, m_s0, m_s1):
            r[...] = neg
        for r in (s_v0, s_v1, w_v0, w_v1, s_s0, s_s1, w_s0, w_s1):
            r[...] = zero

    HB = BLK // 2

    def stream(x, Wl, xr_ref, AE, m_ref, s_ref, w_ref):
        y = _dot(x, Wl)
        z = y + xr_ref[...]
        e = jnp.maximum(z, 0.2 * z)
        lb = _dot(e, AE)
        m_old = m_ref[...]
        m_new = jnp.maximum(m_old, jnp.max(lb, axis=0, keepdims=True))
        corr = jnp.exp2(m_old - m_new)
        pb = jnp.exp2(lb - m_new)
        s_ref[...] = s_ref[...] * corr + jnp.sum(pb, axis=0, keepdims=True)
        w_ref[...] = w_ref[...] * corr + jnp.sum(pb * y, axis=0, keepdims=True)
        m_ref[...] = m_new

    AEv = AE_v[...]
    AEs = AE_s[...]
    Wlv = Wl_v[...]
    Wls = Wl_s[...]
    stream(view_ref[:HB, :], Wlv, xr_v, AEv, m_v0, s_v0, w_v0)
    stream(sp_ref[:HB, :], Wls, xr_s, AEs, m_s0, s_s0, w_s0)
    stream(view_ref[HB:, :], Wlv, xr_v, AEv, m_v1, s_v1, w_v1)
    stream(sp_ref[HB:, :], Wls, xr_s, AEs, m_s1, s_s1, w_s1)

    @pl.when(i == NB - 1)
    def _fin():
        def merge(m0, s0, w0, m1, s1, w1):
            m = jnp.maximum(m0[...], m1[...])
            c0 = jnp.exp2(m0[...] - m)
            c1 = jnp.exp2(m1[...] - m)
            return s0[...] * c0 + s1[...] * c1, w0[...] * c0 + w1[...] * c1

        sv, wv = merge(m_v0, s_v0, w_v0, m_v1, s_v1, w_v1)
        ss, ws = merge(m_s0, s_s0, w_s0, m_s1, s_s1, w_s1)
        v2g = wv / sv + bb_v[...]
        s2g = ws / ss + bb_s[...]
        x = g_ref[...] + jnp.concatenate([v2g, s2g], axis=1)
        y = jnp.maximum(_ln(x, ln_pre_s[...], ln_pre_b[...]), 0.0)
        y = _dot(y, W_mlp[...]) + b_mlp[...]
        out_ref[...] = x + y


def kernel(view_features, scenepoint_features, prev_global_features,
           ln_g2v_s, ln_g2v_b, W_g2v, b_g2v,
           Wl_v, bl_v, Wr_v, br_v, att_v, bias_v,
           ln_g2s_s, ln_g2s_b, W_g2s, b_g2s,
           Wl_s, bl_s, Wr_s, br_s, att_s, bias_s,
           ln_pre_s, ln_pre_b, W_mlp, b_mlp):
    row = lambda a: a.reshape(1, -1)
    heads = jnp.arange(F) // C
    same = (heads[:, None] == heads[None, :]).astype(jnp.float32)
    log2e = 1.4426950408889634
    AE_v = same * (att_v.reshape(-1)[:, None] * log2e)
    AE_s = same * (att_s.reshape(-1)[:, None] * log2e)
    bb_v = row(bl_v + bias_v)
    bb_s = row(bl_s + bias_s)

    blk = pl.BlockSpec((BLK, F), lambda i: (i, 0))

    def full(shape):
        return pl.BlockSpec(shape, lambda i: (0,) * len(shape))

    ins = [
        view_features, scenepoint_features, prev_global_features,
        row(ln_g2v_s), row(ln_g2v_b), W_g2v, row(b_g2v),
        Wl_v, row(bl_v), Wr_v, row(br_v), AE_v, bb_v,
        row(ln_g2s_s), row(ln_g2s_b), W_g2s, row(b_g2s),
        Wl_s, row(bl_s), Wr_s, row(br_s), AE_s, bb_s,
        row(ln_pre_s), row(ln_pre_b), W_mlp, row(b_mlp),
    ]
    in_specs = [blk, blk] + [full(a.shape) for a in ins[2:]]

    scratch = [pltpu.VMEM((1, F), jnp.float32) for _ in range(14)]

    return pl.pallas_call(
        _kernel,
        grid=(NB,),
        in_specs=in_specs,
        out_specs=full((1, FG)),
        out_shape=jax.ShapeDtypeStruct((1, FG), jnp.float32),
        scratch_shapes=scratch,
        compiler_params=pltpu.CompilerParams(
            dimension_semantics=("arbitrary",)),
    )(*ins)
